# Initial kernel scaffold; baseline (speedup 1.0000x reference)
import jax
import jax.numpy as jnp
from jax.experimental import pallas as pl


def kernel(user_linear, item_linear, inter_u, inter_i, pruning):
    raise NotImplementedError("write your pallas kernel here")


# speedup vs baseline: 4.5954x; 4.5954x over previous
import jax
import jax.numpy as jnp
from jax import lax
from jax.experimental import pallas as pl
from jax.experimental.pallas import tpu as pltpu
from jax.experimental.pallas import tpu_sc as plsc

f32 = jnp.float32
i32 = jnp.int32

N_USERS = 25000
N_ITEMS = 25000
NNZ = 800000
D = 64
EPS = 1e-8

NPAD = 25088
RPT = NPAD // 16
DUMP = NPAD - 1
W = 80
SUB = 80
KSUB = 8
RS_PER_TEC = 632
NBLK = RS_PER_TEC // KSUB
E_PER_TEC = RS_PER_TEC * SUB
NNZ_PAD = 16 * E_PER_TEC

_MESH = plsc.VectorSubcoreMesh(core_axis_name="c", subcore_axis_name="s")
_SC_PARAMS = pltpu.CompilerParams(
    use_tc_tiling_on_sc=False, needs_layout_passes=False
)


def _k1(user_linear, item_linear, uix2d, iix2d, z64):
    @pl.kernel(
        out_type=(
            jax.ShapeDtypeStruct((NPAD, D), f32),
            jax.ShapeDtypeStruct((NPAD, D), f32),
        ),
        mesh=_MESH,
        compiler_params=_SC_PARAMS,
        scratch_types=[
            pltpu.VMEM((KSUB, SUB), i32),
            pltpu.VMEM((KSUB, SUB), i32),
            pltpu.VMEM((SUB, D), f32),
            pltpu.VMEM_SHARED((NPAD, D), f32),
            pltpu.SemaphoreType.DMA,
        ],
    )
    def k1(ul, il, uix, iix, zz, fu_out, fi_out, gix, six, rows, acc, gsem):
        core = lax.axis_index("c")
        tec = lax.axis_index("s")
        row0 = tec * RPT
        pltpu.sync_copy(zz.at[pl.ds(row0, RPT)], acc.at[pl.ds(row0, RPT)])
        plsc.subcore_barrier()

        def side(table, g2d, s2d):
            @pl.loop(0, NBLK)
            def _(blk):
                r0 = tec * RS_PER_TEC + blk * KSUB
                pltpu.sync_copy(g2d.at[pl.ds(r0, KSUB)], gix)
                pltpu.sync_copy(s2d.at[pl.ds(r0, KSUB)], six)
                for j in range(KSUB):
                    pltpu.async_copy(table.at[gix.at[j]], rows, gsem).wait()
                    pltpu.sync_copy(rows, acc.at[six.at[j]], add=True)

        @pl.when(core == 0)
        def _():
            side(ul, iix, uix)

        @pl.when(core == 1)
        def _():
            side(il, uix, iix)

        plsc.subcore_barrier()

        @pl.when(core == 0)
        def _():
            pltpu.sync_copy(acc.at[pl.ds(row0, RPT)], fu_out.at[pl.ds(row0, RPT)])

        @pl.when(core == 1)
        def _():
            pltpu.sync_copy(acc.at[pl.ds(row0, RPT)], fi_out.at[pl.ds(row0, RPT)])

    return k1(user_linear, item_linear, uix2d, iix2d, z64)


def _k2(fu_raw, fi_raw):
    def body(a_ref, b_ref, oa_ref, ob_ref):
        for x_ref, o_ref in ((a_ref, oa_ref), (b_ref, ob_ref)):
            x = x_ref[...]
            ss = jnp.sum(x * x, axis=1, keepdims=True)
            r = 1.0 / jnp.maximum(jnp.sqrt(ss), EPS)
            o_ref[...] = jnp.concatenate(
                [x, jnp.broadcast_to(r, (x.shape[0], 16))], axis=1
            )

    return pl.pallas_call(
        body,
        grid=(NPAD // 512,),
        in_specs=[pl.BlockSpec((512, D), lambda g: (g, 0))] * 2,
        out_specs=[pl.BlockSpec((512, W), lambda g: (g, 0))] * 2,
        out_shape=[jax.ShapeDtypeStruct((NPAD, W), f32)] * 2,
    )(fu_raw, fi_raw)


def _k3(fu_pad, fi_pad, uix2d, iix2d, z64, prun_vec):
    @pl.kernel(
        out_type=(
            jax.ShapeDtypeStruct((NPAD, D), f32),
            jax.ShapeDtypeStruct((NPAD, D), f32),
            jax.ShapeDtypeStruct((NNZ_PAD, 16), f32),
        ),
        mesh=_MESH,
        compiler_params=_SC_PARAMS,
        scratch_types=[
            pltpu.VMEM((KSUB, SUB), i32),
            pltpu.VMEM((KSUB, SUB), i32),
            pltpu.VMEM((SUB, W), f32),
            pltpu.VMEM((SUB, W), f32),
            pltpu.VMEM((SUB, D), f32),
            pltpu.VMEM((SUB, 16), f32),
            pltpu.VMEM((16,), f32),
            pltpu.VMEM_SHARED((NPAD, D), f32),
            pltpu.SemaphoreType.DMA,
        ],
    )
    def k3(fu, fi, uix, iix, zz, pv_hbm, u1_out, i1_out, v_out,
           uixv, iixv, rows_u, rows_i, outb, vb, pvv, acc, gsem):
        core = lax.axis_index("c")
        tec = lax.axis_index("s")
        row0 = tec * RPT
        pltpu.sync_copy(zz.at[pl.ds(row0, RPT)], acc.at[pl.ds(row0, RPT)])
        pltpu.sync_copy(pv_hbm, pvv)
        plsc.subcore_barrier()
        pv = pvv[...]

        def side(scale_item_rows, write_v):
            @pl.loop(0, NBLK)
            def _(blk):
                r0 = tec * RS_PER_TEC + blk * KSUB
                pltpu.sync_copy(uix.at[pl.ds(r0, KSUB)], uixv)
                pltpu.sync_copy(iix.at[pl.ds(r0, KSUB)], iixv)
                for j in range(KSUB):
                    cu = pltpu.async_copy(fu.at[uixv.at[j]], rows_u, gsem)
                    ci = pltpu.async_copy(fi.at[iixv.at[j]], rows_i, gsem)
                    cu.wait()
                    ci.wait()

                    @pl.loop(0, SUB)
                    def _(e):
                        fuv = [rows_u[e, pl.ds(16 * k, 16)] for k in range(4)]
                        fiv = [rows_i[e, pl.ds(16 * k, 16)] for k in range(4)]
                        p = (fuv[0] * fiv[0] + fuv[1] * fiv[1]
                             + fuv[2] * fiv[2] + fuv[3] * fiv[3])
                        s = jnp.sum(p)
                        ru = rows_u[e, pl.ds(64, 16)]
                        ri = rows_i[e, pl.ds(64, 16)]
                        sim = (s * (ru * ri) + 1.0) * 0.5
                        v = jnp.where(sim < pv, 0.0, sim)
                        other = fiv if scale_item_rows else fuv
                        for k in range(4):
                            outb[e, pl.ds(16 * k, 16)] = other[k] * v
                        if write_v:
                            vb[e, pl.ds(0, 16)] = v

                    sixv = uixv if scale_item_rows else iixv
                    pltpu.sync_copy(outb, acc.at[sixv.at[j]], add=True)
                    if write_v:
                        eoff = (r0 + j) * SUB
                        pltpu.sync_copy(vb, v_out.at[pl.ds(eoff, SUB)])

        @pl.when(core == 0)
        def _():
            side(True, True)

        @pl.when(core == 1)
        def _():
            side(False, False)

        plsc.subcore_barrier()

        @pl.when(core == 0)
        def _():
            pltpu.sync_copy(acc.at[pl.ds(row0, RPT)], u1_out.at[pl.ds(row0, RPT)])

        @pl.when(core == 1)
        def _():
            pltpu.sync_copy(acc.at[pl.ds(row0, RPT)], i1_out.at[pl.ds(row0, RPT)])

    return k3(fu_pad, fi_pad, uix2d, iix2d, z64, prun_vec)


def _k3b(vtab, uix2d, iix2d, z16):
    @pl.kernel(
        out_type=(
            jax.ShapeDtypeStruct((NPAD, 16), f32),
            jax.ShapeDtypeStruct((NPAD, 16), f32),
        ),
        mesh=_MESH,
        compiler_params=_SC_PARAMS,
        scratch_types=[
            pltpu.VMEM((KSUB, SUB), i32),
            pltpu.VMEM((SUB, 16), f32),
            pltpu.VMEM_SHARED((NPAD, 16), f32),
            pltpu.SemaphoreType.DMA,
        ],
    )
    def k3b(vt, uix, iix, zz, du_out, di_out, six, vb, acc, gsem):
        core = lax.axis_index("c")
        tec = lax.axis_index("s")
        row0 = tec * RPT
        pltpu.sync_copy(zz.at[pl.ds(row0, RPT)], acc.at[pl.ds(row0, RPT)])
        plsc.subcore_barrier()

        def side(s2d):
            @pl.loop(0, NBLK)
            def _(blk):
                r0 = tec * RS_PER_TEC + blk * KSUB
                pltpu.sync_copy(s2d.at[pl.ds(r0, KSUB)], six)
                for j in range(KSUB):
                    eoff = (r0 + j) * SUB
                    pltpu.sync_copy(vt.at[pl.ds(eoff, SUB)], vb)
                    pltpu.sync_copy(vb, acc.at[six.at[j]], add=True)

        @pl.when(core == 0)
        def _():
            side(uix)

        @pl.when(core == 1)
        def _():
            side(iix)

        plsc.subcore_barrier()

        @pl.when(core == 0)
        def _():
            pltpu.sync_copy(acc.at[pl.ds(row0, RPT)], du_out.at[pl.ds(row0, RPT)])

        @pl.when(core == 1)
        def _():
            pltpu.sync_copy(acc.at[pl.ds(row0, RPT)], di_out.at[pl.ds(row0, RPT)])

    return k3b(vtab, uix2d, iix2d, z16)


def _k4(u1, i1, du, di):
    def body(a_ref, b_ref, da_ref, db_ref, oa_ref, ob_ref):
        for x_ref, d_ref, o_ref in ((a_ref, da_ref, oa_ref),
                                    (b_ref, db_ref, ob_ref)):
            o_ref[...] = x_ref[...] / (d_ref[:, 0:1] + 1e-7)

    spec64 = pl.BlockSpec((512, D), lambda g: (g, 0))
    spec16 = pl.BlockSpec((512, 16), lambda g: (g, 0))
    return pl.pallas_call(
        body,
        grid=(NPAD // 512,),
        in_specs=[spec64, spec64, spec16, spec16],
        out_specs=[spec64] * 2,
        out_shape=[jax.ShapeDtypeStruct((NPAD, D), f32)] * 2,
    )(u1, i1, du, di)


def _k5(e1u, e1i, uix2d, iix2d, vtab, z64):
    @pl.kernel(
        out_type=(
            jax.ShapeDtypeStruct((NPAD, D), f32),
            jax.ShapeDtypeStruct((NPAD, D), f32),
        ),
        mesh=_MESH,
        compiler_params=_SC_PARAMS,
        scratch_types=[
            pltpu.VMEM((KSUB, SUB), i32),
            pltpu.VMEM((KSUB, SUB), i32),
            pltpu.VMEM((SUB, D), f32),
            pltpu.VMEM((SUB, 16), f32),
            pltpu.VMEM_SHARED((NPAD, D), f32),
            pltpu.SemaphoreType.DMA,
        ],
    )
    def k5(eu, ei, uix, iix, vt, zz, u2_out, i2_out,
           gix, six, rows, vb, acc, gsem):
        core = lax.axis_index("c")
        tec = lax.axis_index("s")
        row0 = tec * RPT
        pltpu.sync_copy(zz.at[pl.ds(row0, RPT)], acc.at[pl.ds(row0, RPT)])
        plsc.subcore_barrier()

        def side(table, g2d, s2d):
            @pl.loop(0, NBLK)
            def _(blk):
                r0 = tec * RS_PER_TEC + blk * KSUB
                pltpu.sync_copy(g2d.at[pl.ds(r0, KSUB)], gix)
                pltpu.sync_copy(s2d.at[pl.ds(r0, KSUB)], six)
                for j in range(KSUB):
                    eoff = (r0 + j) * SUB
                    cg = pltpu.async_copy(table.at[gix.at[j]], rows, gsem)
                    pltpu.sync_copy(vt.at[pl.ds(eoff, SUB)], vb)
                    cg.wait()

                    @pl.loop(0, SUB)
                    def _(e):
                        v = vb[e, pl.ds(0, 16)]
                        for k in range(4):
                            rows[e, pl.ds(16 * k, 16)] = rows[e, pl.ds(16 * k, 16)] * v

                    pltpu.sync_copy(rows, acc.at[six.at[j]], add=True)

        @pl.when(core == 0)
        def _():
            side(ei, iix, uix)

        @pl.when(core == 1)
        def _():
            side(eu, uix, iix)

        plsc.subcore_barrier()

        @pl.when(core == 0)
        def _():
            pltpu.sync_copy(acc.at[pl.ds(row0, RPT)], u2_out.at[pl.ds(row0, RPT)])

        @pl.when(core == 1)
        def _():
            pltpu.sync_copy(acc.at[pl.ds(row0, RPT)], i2_out.at[pl.ds(row0, RPT)])

    return k5(e1u, e1i, uix2d, iix2d, vtab, z64)


def _k6(fu_raw, e1u, u2, du, fi_raw, e1i, i2, di):
    def body(f_a, e_a, n_a, d_a, f_b, e_b, n_b, d_b, o_a, o_b):
        for f_ref, e_ref, n_ref, d_ref, o_ref in (
            (f_a, e_a, n_a, d_a, o_a),
            (f_b, e_b, n_b, d_b, o_b),
        ):
            d = d_ref[:, 0:1] + 1e-7
            o_ref[...] = (f_ref[...] + e_ref[...] + n_ref[...] / d) * (1.0 / 3.0)

    spec64 = pl.BlockSpec((512, D), lambda g: (g, 0))
    spec16 = pl.BlockSpec((512, 16), lambda g: (g, 0))
    return pl.pallas_call(
        body,
        grid=(NPAD // 512,),
        in_specs=[spec64, spec64, spec64, spec16] * 2,
        out_specs=[spec64] * 2,
        out_shape=[jax.ShapeDtypeStruct((NPAD, D), f32)] * 2,
    )(fu_raw, e1u, u2, du, fi_raw, e1i, i2, di)


def kernel(user_linear, item_linear, inter_u, inter_i, pruning):
    pad = jnp.full((NNZ_PAD - NNZ,), DUMP, i32)
    uix2d = jnp.concatenate([inter_u.astype(i32), pad]).reshape(NNZ_PAD // SUB, SUB)
    iix2d = jnp.concatenate([inter_i.astype(i32), pad]).reshape(NNZ_PAD // SUB, SUB)
    prun_f = jnp.asarray(pruning, f32)
    prun_eff = jnp.where(prun_f > 0, prun_f, f32(-jnp.inf))
    prun_vec = jnp.full((16,), prun_eff, f32)
    z64 = jnp.zeros((NPAD, D), f32)
    z16 = jnp.zeros((NPAD, 16), f32)
    ul_pad = jnp.pad(user_linear, ((0, NPAD - N_ITEMS), (0, 0)))
    il_pad = jnp.pad(item_linear, ((0, NPAD - N_USERS), (0, 0)))

    fu_raw, fi_raw = _k1(ul_pad, il_pad, uix2d, iix2d, z64)
    fu_pad, fi_pad = _k2(fu_raw, fi_raw)
    u1, i1, vtab = _k3(fu_pad, fi_pad, uix2d, iix2d, z64, prun_vec)
    du, di = _k3b(vtab, uix2d, iix2d, z16)
    e1u, e1i = _k4(u1, i1, du, di)
    u2, i2 = _k5(e1u, e1i, uix2d, iix2d, vtab, z64)
    uo, io = _k6(fu_raw, e1u, u2, du, fi_raw, e1i, i2, di)
    return uo[:N_USERS], io[:N_ITEMS]
